# baseline (device time: 201230 ns/iter reference)
import jax
import jax.numpy as jnp
from jax import lax
from jax.experimental import pallas as pl
from jax.experimental.pallas import tpu as pltpu

N_DEV = 32
M = 4096
M_PER = M // N_DEV
N_COLS = 2048
N_HALF = N_COLS // 2
NSB = 4
N_SUB = N_HALF // NSB
N_STEPS = N_DEV - 1
LOG2_DEV = 5


def _ring_tables():
    logical = []
    for z in range(4):
        for y in range(4):
            for x in ((0, 1) if y % 2 == 0 else (1, 0)):
                logical.append((x, y, z))
    bous = [(0, 0), (1, 0), (2, 0), (3, 0), (3, 1), (2, 1), (1, 1), (0, 1),
            (0, 2), (1, 2), (2, 2), (3, 2), (3, 3), (2, 3), (1, 3), (0, 3)]
    cycle = [(0, y, z) for (y, z) in bous] + \
            [(1, y, z) for (y, z) in reversed(bous)]
    sigma = [logical.index(c) for c in cycle]
    inv = [0] * N_DEV
    for r, l in enumerate(sigma):
        inv[l] = r
    flips = [(1, 0, 0), (0, 1, 0), (0, 2, 0), (0, 0, 1), (0, 0, 2)]
    pbit = []
    for fx, fy, fz in flips:
        row = []
        for (cx, cy, cz) in logical:
            row.append(logical.index((cx ^ fx, cy ^ fy, cz ^ fz)))
        pbit.append(row)
    return sigma, inv, pbit


_SIGMA, _INV, _PBIT = _ring_tables()


def kernel(x, w_mat):
    def body(sigma_ref, inv_ref, pbit_ref, x_ref, w_ref, out_ref,
             send_r, recv_r, send_l, recv_l,
             send_sems_r, recv_sems_r, send_sems_l, recv_sems_l,
             credit_r, credit_l,
             amax_send, amax_recv, amax_send_sems, amax_recv_sems):
        d = lax.axis_index("i")
        rho = inv_ref[d]
        right = sigma_ref[jnp.mod(rho + 1, N_DEV)]
        left = sigma_ref[jnp.mod(rho - 1, N_DEV)]

        barrier_sem = pltpu.get_barrier_semaphore()
        for nbr in (left, right):
            pl.semaphore_signal(barrier_sem, inc=1, device_id=(nbr,),
                                device_id_type=pl.DeviceIdType.MESH)
        pl.semaphore_wait(barrier_sem, 2)

        def partial(c, lo):
            return jnp.dot(
                x_ref[pl.ds(c * M_PER, M_PER), :],
                w_ref[:, lo:lo + N_SUB],
                preferred_element_type=jnp.float32,
            )

        dirs = {
            "r": (send_r, recv_r, send_sems_r, recv_sems_r, right, left, 0),
            "l": (send_l, recv_l, send_sems_l, recv_sems_l, left, right, N_HALF),
        }
        credit_arrs = {"r": credit_r, "l": credit_l}

        def chunk_id(dirname, s):
            if dirname == "r":
                return sigma_ref[jnp.mod(rho - 2 - s, N_DEV)]
            return sigma_ref[jnp.mod(rho + 2 + s, N_DEV)]

        def make_rdma(dirname, sub, slot):
            sb, rb, ss, rs, peer_out, _, _ = dirs[dirname]
            return pltpu.make_async_remote_copy(
                src_ref=sb.at[sub, slot], dst_ref=rb.at[sub, slot],
                send_sem=ss.at[sub, slot], recv_sem=rs.at[sub, slot],
                device_id=(peer_out,), device_id_type=pl.DeviceIdType.MESH)

        last_send = {}

        c0 = {"r": sigma_ref[jnp.mod(rho - 1, N_DEV)],
              "l": sigma_ref[jnp.mod(rho + 1, N_DEV)]}
        for sub in range(NSB):
            for dirname in ("r", "l"):
                sb = dirs[dirname][0]
                base = dirs[dirname][6]
                sb[sub, 0, :, :] = partial(c0[dirname], base + sub * N_SUB)
                rd = make_rdma(dirname, sub, 0)
                rd.start()
                last_send[(dirname, sub, 0)] = rd

        y = {}
        for s in range(N_STEPS):
            sp = s % 2
            nsp = (s + 1) % 2
            last = s == N_STEPS - 1
            p = {}
            for dirname in ("r", "l"):
                base = dirs[dirname][6]
                c = chunk_id(dirname, s)
                for sub in range(NSB):
                    p[(dirname, sub)] = partial(c, base + sub * N_SUB)

            for sub in range(NSB):
                for dirname in ("r", "l"):
                    sb, rb, ss, rs, peer_out, peer_in, base = dirs[dirname]
                    make_rdma(dirname, sub, sp).wait_recv()
                    acc = rb[sub, sp, :, :] + p[(dirname, sub)]
                    if not last:
                        prev = last_send.get((dirname, sub, nsp))
                        if prev is not None:
                            prev.wait_send()
                        sb[sub, nsp, :, :] = acc
                        if s >= 1:
                            pl.semaphore_wait(credit_arrs[dirname].at[sub], 1)
                        rd = make_rdma(dirname, sub, nsp)
                        rd.start()
                        last_send[(dirname, sub, nsp)] = rd
                        if s <= N_STEPS - 3:
                            pl.semaphore_signal(
                                credit_arrs[dirname].at[sub], inc=1,
                                device_id=(peer_in,),
                                device_id_type=pl.DeviceIdType.MESH)
                    else:
                        y[(dirname, sub)] = jnp.maximum(acc, 0.0)

        for key, rd in last_send.items():
            rd.wait_send()

        amax = jnp.max(jnp.stack([jnp.max(v) for v in y.values()]))
        for r in range(LOG2_DEV):
            partner = pbit_ref[r, d]
            amax_send[r, :, :] = jnp.full((8, 128), amax, jnp.float32)
            ex = pltpu.make_async_remote_copy(
                src_ref=amax_send.at[r], dst_ref=amax_recv.at[r],
                send_sem=amax_send_sems.at[r], recv_sem=amax_recv_sems.at[r],
                device_id=(partner,), device_id_type=pl.DeviceIdType.MESH)
            ex.start()
            ex.wait()
            amax = jnp.maximum(amax, jnp.max(amax_recv[r, :, :]))

        scale = amax / 127.0
        inv_scale = 127.0 / amax
        for (dirname, sub), yv in y.items():
            base = dirs[dirname][6] + sub * N_SUB
            q = jnp.clip(jnp.round(yv * inv_scale), 0.0, 127.0)
            out_ref[:, base:base + N_SUB] = q * scale

    sigma_arr = jnp.array(_SIGMA, dtype=jnp.int32)
    inv_arr = jnp.array(_INV, dtype=jnp.int32)
    pbit_arr = jnp.array(_PBIT, dtype=jnp.int32)

    return pl.pallas_call(
        body,
        out_shape=jax.ShapeDtypeStruct((M_PER, N_COLS), jnp.float32),
        in_specs=[
            pl.BlockSpec(memory_space=pltpu.SMEM),
            pl.BlockSpec(memory_space=pltpu.SMEM),
            pl.BlockSpec(memory_space=pltpu.SMEM),
            pl.BlockSpec(memory_space=pltpu.VMEM),
            pl.BlockSpec(memory_space=pltpu.VMEM),
        ],
        out_specs=pl.BlockSpec(memory_space=pltpu.VMEM),
        scratch_shapes=[
            pltpu.VMEM((NSB, 2, M_PER, N_SUB), jnp.float32),
            pltpu.VMEM((NSB, 2, M_PER, N_SUB), jnp.float32),
            pltpu.VMEM((NSB, 2, M_PER, N_SUB), jnp.float32),
            pltpu.VMEM((NSB, 2, M_PER, N_SUB), jnp.float32),
            pltpu.SemaphoreType.DMA((NSB, 2)),
            pltpu.SemaphoreType.DMA((NSB, 2)),
            pltpu.SemaphoreType.DMA((NSB, 2)),
            pltpu.SemaphoreType.DMA((NSB, 2)),
            pltpu.SemaphoreType.REGULAR((NSB,)),
            pltpu.SemaphoreType.REGULAR((NSB,)),
            pltpu.VMEM((LOG2_DEV, 8, 128), jnp.float32),
            pltpu.VMEM((LOG2_DEV, 8, 128), jnp.float32),
            pltpu.SemaphoreType.DMA((LOG2_DEV,)),
            pltpu.SemaphoreType.DMA((LOG2_DEV,)),
        ],
        compiler_params=pltpu.CompilerParams(collective_id=0),
    )(sigma_arr, inv_arr, pbit_arr, x, w_mat)


# device time: 200654 ns/iter; 1.0029x vs baseline; 1.0029x over previous
import jax
import jax.numpy as jnp
from jax import lax
from jax.experimental import pallas as pl
from jax.experimental.pallas import tpu as pltpu

N_DEV = 32
M = 4096
M_PER = M // N_DEV
N_COLS = 2048
N_HALF = N_COLS // 2
NSB = 4
N_SUB = N_HALF // NSB
N_STEPS = N_DEV - 1
LOG2_DEV = 5


def _ring_tables():
    logical = []
    for z in range(4):
        for y in range(4):
            for x in ((0, 1) if y % 2 == 0 else (1, 0)):
                logical.append((x, y, z))
    bous = [(0, 0), (1, 0), (2, 0), (3, 0), (3, 1), (2, 1), (1, 1), (0, 1),
            (0, 2), (1, 2), (2, 2), (3, 2), (3, 3), (2, 3), (1, 3), (0, 3)]
    cycle = [(0, y, z) for (y, z) in bous] + \
            [(1, y, z) for (y, z) in reversed(bous)]
    sigma = [logical.index(c) for c in cycle]
    inv = [0] * N_DEV
    for r, l in enumerate(sigma):
        inv[l] = r
    zt = [c[2] for c in logical]
    pt = [c[0] * 4 + c[1] for c in logical]
    st1 = [[logical.index((cx, cy, j)) for j in range(4)]
           for (cx, cy, cz) in logical]
    st2 = [[logical.index((j // 4, j % 4, cz)) for j in range(8)]
           for (cx, cy, cz) in logical]
    return sigma, inv, zt, pt, st1, st2


_SIGMA, _INV, _ZT, _PT, _ST1, _ST2 = _ring_tables()


def kernel(x, w_mat):
    def body(sigma_ref, inv_ref, zt_ref, pt_ref, st1_ref, st2_ref,
             x_ref, w_ref, out_ref,
             send_r, recv_r, send_l, recv_l,
             send_sems_r, recv_sems_r, send_sems_l, recv_sems_l,
             credit_r, credit_l,
             a1_send, a1_recv, a1_send_sems, a1_recv_sems,
             a2_send, a2_recv, a2_send_sems, a2_recv_sems):
        d = lax.axis_index("i")
        rho = inv_ref[d]
        right = sigma_ref[jnp.mod(rho + 1, N_DEV)]
        left = sigma_ref[jnp.mod(rho - 1, N_DEV)]

        barrier_sem = pltpu.get_barrier_semaphore()
        for nbr in (left, right):
            pl.semaphore_signal(barrier_sem, inc=1, device_id=(nbr,),
                                device_id_type=pl.DeviceIdType.MESH)
        pl.semaphore_wait(barrier_sem, 2)

        def partial(c, lo):
            return jnp.dot(
                x_ref[pl.ds(c * M_PER, M_PER), :],
                w_ref[:, lo:lo + N_SUB],
                preferred_element_type=jnp.float32,
            )

        dirs = {
            "r": (send_r, recv_r, send_sems_r, recv_sems_r, right, left, 0),
            "l": (send_l, recv_l, send_sems_l, recv_sems_l, left, right, N_HALF),
        }
        credit_arrs = {"r": credit_r, "l": credit_l}

        def chunk_id(dirname, s):
            if dirname == "r":
                return sigma_ref[jnp.mod(rho - 2 - s, N_DEV)]
            return sigma_ref[jnp.mod(rho + 2 + s, N_DEV)]

        def make_rdma(dirname, sub, slot):
            sb, rb, ss, rs, peer_out, _, _ = dirs[dirname]
            return pltpu.make_async_remote_copy(
                src_ref=sb.at[sub, slot], dst_ref=rb.at[sub, slot],
                send_sem=ss.at[sub, slot], recv_sem=rs.at[sub, slot],
                device_id=(peer_out,), device_id_type=pl.DeviceIdType.MESH)

        last_send = {}

        c0 = {"r": sigma_ref[jnp.mod(rho - 1, N_DEV)],
              "l": sigma_ref[jnp.mod(rho + 1, N_DEV)]}
        for sub in range(NSB):
            for dirname in ("r", "l"):
                sb = dirs[dirname][0]
                base = dirs[dirname][6]
                sb[sub, 0, :, :] = partial(c0[dirname], base + sub * N_SUB)
                rd = make_rdma(dirname, sub, 0)
                rd.start()
                last_send[(dirname, sub, 0)] = rd

        y = {}
        for s in range(N_STEPS):
            sp = s % 2
            nsp = (s + 1) % 2
            last = s == N_STEPS - 1
            p = {}
            for dirname in ("r", "l"):
                base = dirs[dirname][6]
                c = chunk_id(dirname, s)
                for sub in range(NSB):
                    p[(dirname, sub)] = partial(c, base + sub * N_SUB)

            for sub in range(NSB):
                for dirname in ("r", "l"):
                    sb, rb, ss, rs, peer_out, peer_in, base = dirs[dirname]
                    make_rdma(dirname, sub, sp).wait_recv()
                    acc = rb[sub, sp, :, :] + p[(dirname, sub)]
                    if not last:
                        prev = last_send.get((dirname, sub, nsp))
                        if prev is not None:
                            prev.wait_send()
                        sb[sub, nsp, :, :] = acc
                        if s >= 1:
                            pl.semaphore_wait(credit_arrs[dirname].at[sub], 1)
                        rd = make_rdma(dirname, sub, nsp)
                        rd.start()
                        last_send[(dirname, sub, nsp)] = rd
                        if s <= N_STEPS - 3:
                            pl.semaphore_signal(
                                credit_arrs[dirname].at[sub], inc=1,
                                device_id=(peer_in,),
                                device_id_type=pl.DeviceIdType.MESH)
                    else:
                        y[(dirname, sub)] = jnp.maximum(acc, 0.0)

        for key, rd in last_send.items():
            rd.wait_send()

        amax = jnp.max(jnp.stack([jnp.max(v) for v in y.values()]))

        def a2a(send_buf, recv_buf, send_sems, recv_sems, group_ref,
                n_group, my_slot, value):
            send_buf[:, :] = jnp.full((8, 128), value, jnp.float32)
            recv_buf[my_slot, :, :] = jnp.full((8, 128), value, jnp.float32)
            for j in range(n_group):
                tgt = group_ref[d, j]
                snd = pltpu.make_async_remote_copy(
                    src_ref=send_buf, dst_ref=recv_buf.at[my_slot],
                    send_sem=send_sems.at[j], recv_sem=recv_sems.at[my_slot],
                    device_id=(tgt,), device_id_type=pl.DeviceIdType.MESH)
                pl.when(tgt != d)(snd.start)
            for j in range(n_group):
                rcv = pltpu.make_async_remote_copy(
                    src_ref=send_buf, dst_ref=recv_buf.at[j],
                    send_sem=send_sems.at[j], recv_sem=recv_sems.at[j],
                    device_id=(d,), device_id_type=pl.DeviceIdType.MESH)
                pl.when(j != my_slot)(rcv.wait_recv)
            result = jnp.max(recv_buf[:, :, :])
            for j in range(n_group):
                tgt = group_ref[d, j]
                snd = pltpu.make_async_remote_copy(
                    src_ref=send_buf, dst_ref=recv_buf.at[my_slot],
                    send_sem=send_sems.at[j], recv_sem=recv_sems.at[my_slot],
                    device_id=(tgt,), device_id_type=pl.DeviceIdType.MESH)
                pl.when(tgt != d)(snd.wait_send)
            return result

        amax = a2a(a1_send, a1_recv, a1_send_sems, a1_recv_sems,
                   st1_ref, 4, zt_ref[d], amax)
        amax = a2a(a2_send, a2_recv, a2_send_sems, a2_recv_sems,
                   st2_ref, 8, pt_ref[d], amax)

        scale = amax / 127.0
        inv_scale = 127.0 / amax
        for (dirname, sub), yv in y.items():
            base = dirs[dirname][6] + sub * N_SUB
            q = jnp.clip(jnp.round(yv * inv_scale), 0.0, 127.0)
            out_ref[:, base:base + N_SUB] = q * scale

    sigma_arr = jnp.array(_SIGMA, dtype=jnp.int32)
    inv_arr = jnp.array(_INV, dtype=jnp.int32)
    zt_arr = jnp.array(_ZT, dtype=jnp.int32)
    pt_arr = jnp.array(_PT, dtype=jnp.int32)
    st1_arr = jnp.array(_ST1, dtype=jnp.int32)
    st2_arr = jnp.array(_ST2, dtype=jnp.int32)

    return pl.pallas_call(
        body,
        out_shape=jax.ShapeDtypeStruct((M_PER, N_COLS), jnp.float32),
        in_specs=[
            pl.BlockSpec(memory_space=pltpu.SMEM),
            pl.BlockSpec(memory_space=pltpu.SMEM),
            pl.BlockSpec(memory_space=pltpu.SMEM),
            pl.BlockSpec(memory_space=pltpu.SMEM),
            pl.BlockSpec(memory_space=pltpu.SMEM),
            pl.BlockSpec(memory_space=pltpu.SMEM),
            pl.BlockSpec(memory_space=pltpu.VMEM),
            pl.BlockSpec(memory_space=pltpu.VMEM),
        ],
        out_specs=pl.BlockSpec(memory_space=pltpu.VMEM),
        scratch_shapes=[
            pltpu.VMEM((NSB, 2, M_PER, N_SUB), jnp.float32),
            pltpu.VMEM((NSB, 2, M_PER, N_SUB), jnp.float32),
            pltpu.VMEM((NSB, 2, M_PER, N_SUB), jnp.float32),
            pltpu.VMEM((NSB, 2, M_PER, N_SUB), jnp.float32),
            pltpu.SemaphoreType.DMA((NSB, 2)),
            pltpu.SemaphoreType.DMA((NSB, 2)),
            pltpu.SemaphoreType.DMA((NSB, 2)),
            pltpu.SemaphoreType.DMA((NSB, 2)),
            pltpu.SemaphoreType.REGULAR((NSB,)),
            pltpu.SemaphoreType.REGULAR((NSB,)),
            pltpu.VMEM((8, 128), jnp.float32),
            pltpu.VMEM((4, 8, 128), jnp.float32),
            pltpu.SemaphoreType.DMA((4,)),
            pltpu.SemaphoreType.DMA((4,)),
            pltpu.VMEM((8, 128), jnp.float32),
            pltpu.VMEM((8, 8, 128), jnp.float32),
            pltpu.SemaphoreType.DMA((8,)),
            pltpu.SemaphoreType.DMA((8,)),
        ],
        compiler_params=pltpu.CompilerParams(collective_id=0),
    )(sigma_arr, inv_arr, zt_arr, pt_arr, st1_arr, st2_arr, x, w_mat)
